# baseline (speedup 1.0000x reference)
import jax
import jax.numpy as jnp
from jax import lax
from jax.experimental import pallas as pl
from jax.experimental.pallas import tpu as pltpu
from jax.experimental.pallas import tpu_sc as plsc

_N = 64
_F = 16
_BG = 8


def _sc_body(e_hbm, c_hbm, deg_hbm, e_v, c_v, deg_v):
    cid = lax.axis_index("c")
    sid = lax.axis_index("s")

    @pl.when((cid == 0) & (sid == 0))
    def _():
        E = e_hbm.shape[1]
        pltpu.sync_copy(e_hbm, e_v)
        zeros = jnp.zeros((16,), jnp.float32)
        ones = jnp.ones((16,), jnp.float32)
        lane = lax.broadcasted_iota(jnp.int32, (16,), 0)

        def _zero(k, carry):
            c_v[pl.ds(k * 16, 16)] = zeros
            return carry
        lax.fori_loop(0, 4096 // 16, _zero, 0)
        for q in range(4):
            deg_v[pl.ds(q * 16, 16)] = zeros

        def _edges(t, carry):
            s = e_v[0, pl.ds(t * 16, 16)]
            d = e_v[1, pl.ds(t * 16, 16)]
            f = d * _N + s
            for j in range(16):
                m = lane == j
                plsc.addupdate_scatter(c_v, [f], ones, mask=m)
                plsc.addupdate_scatter(deg_v, [d], ones, mask=m)
            return carry
        lax.fori_loop(0, E // 16, _edges, 0)
        pltpu.sync_copy(c_v, c_hbm)
        pltpu.sync_copy(deg_v, deg_hbm)


def _sc_build(edge_index):
    import functools
    mesh = plsc.VectorSubcoreMesh(core_axis_name="c", subcore_axis_name="s")
    k = functools.partial(
        pl.kernel, mesh=mesh,
        out_type=(jax.ShapeDtypeStruct((_N * _N,), jnp.float32),
                  jax.ShapeDtypeStruct((_N,), jnp.float32)),
        scratch_types=[pltpu.VMEM(edge_index.shape, jnp.int32),
                       pltpu.VMEM((_N * _N,), jnp.float32),
                       pltpu.VMEM((_N,), jnp.float32)],
        compiler_params=pltpu.CompilerParams(needs_layout_passes=False),
    )(_sc_body)
    return k(edge_index)


def _layer(Ab, xin, W, b, rW, rb, g, be):
    mix = jnp.dot(Ab, xin, preferred_element_type=jnp.float32)
    h = (jnp.dot(mix, W, preferred_element_type=jnp.float32) + b
         + jnp.dot(xin, rW, preferred_element_type=jnp.float32) + rb)
    mu = jnp.mean(h, axis=1, keepdims=True)
    var = jnp.mean((h - mu) ** 2, axis=1, keepdims=True)
    hn = (h - mu) * lax.rsqrt(var + 1e-5) * g + be
    return jnp.where(hn > 0, hn, 0.01 * hn)


def _body(x_ref, c_ref, dr_ref, W1r, b1r, W2r, b2r, W3r, b3r,
          rW1r, rb1r, rW2r, rb2r, rW3r, rb3r, fcWr, fcbr,
          g1r, be1r, g2r, be2r, g3r, be3r, gor, bor, unit_ref,
          out_ref, Abig_ref):
    R = _BG * _N
    i = pl.program_id(0)

    @pl.when(i == 0)
    def _build():
        f32 = jnp.float32
        r64 = lax.broadcasted_iota(jnp.int32, (_N, _N), 0)
        c64 = lax.broadcasted_iota(jnp.int32, (_N, _N), 1)
        eye = (r64 == c64).astype(f32)
        C = c_ref[...] + eye
        deg_r = dr_ref[...] + 1.0
        deg_c = jnp.sum(C, axis=1, keepdims=True)
        A = C * lax.rsqrt(deg_c) * lax.rsqrt(deg_r)
        p0 = lax.broadcasted_iota(jnp.int32, (R, _N), 0)
        i1 = lax.broadcasted_iota(jnp.int32, (R, _N), 1)
        E2 = ((p0 & (_N - 1)) == i1).astype(f32)
        i2 = lax.broadcasted_iota(jnp.int32, (_N, R), 0)
        q1 = lax.broadcasted_iota(jnp.int32, (_N, R), 1)
        E2T = (i2 == (q1 & (_N - 1))).astype(f32)
        pg = lax.broadcasted_iota(jnp.int32, (R, R), 0) >> 6
        qg = lax.broadcasted_iota(jnp.int32, (R, R), 1) >> 6
        mask = (pg == qg).astype(f32)
        Abig_ref[...] = jnp.dot(jnp.dot(E2, A, preferred_element_type=f32),
                                E2T, preferred_element_type=f32) * mask

    Ab = Abig_ref[...]
    x = x_ref[...]
    x1 = _layer(Ab, x, W1r[...], b1r[...], rW1r[...], rb1r[...],
                g1r[...], be1r[...])
    x2 = _layer(Ab, x1, W2r[...], b2r[...], rW2r[...], rb2r[...],
                g2r[...], be2r[...])
    x3 = _layer(Ab, x2, W3r[...], b3r[...], rW3r[...], rb3r[...],
                g3r[...], be3r[...])
    h4 = (jnp.dot(x3, fcWr[0:256, :], preferred_element_type=jnp.float32)
          + jnp.dot(x, fcWr[256:272, :], preferred_element_type=jnp.float32)
          + fcbr[...])
    mu = jnp.mean(h4, axis=1, keepdims=True)
    var = jnp.mean((h4 - mu) ** 2, axis=1, keepdims=True)
    hn = (h4 - mu) * lax.rsqrt(var + 1e-5) * gor[...] + bor[...]
    y = jnp.tanh(hn) * unit_ref[...]
    out_ref[...] = y


def kernel(state, edge_index, batch_size, rej_rate, theta_value,
           W1, b1, W2, b2, W3, b3, rW1, rb1, rW2, rb2, rW3, rb3, fcW, fcb,
           g1, be1, g2, be2, g3, be3, go, bo):
    B = state.shape[0] // _N
    R = _BG * _N
    grid = B // _BG
    f32 = jnp.float32
    c_flat, deg = _sc_build(edge_index.astype(jnp.int32))
    c2d = c_flat.reshape(_N, _N)
    deg_row = deg.reshape(1, _N)
    unit = (jnp.asarray(batch_size).astype(f32) / jnp.asarray(B, f32)
            ).reshape(1, 1)
    row = lambda v: v.reshape(1, -1).astype(f32)
    full = lambda a: pl.BlockSpec(a.shape, lambda i: (0,) * a.ndim)
    args = (c2d, deg_row, W1, row(b1), W2, row(b2), W3, row(b3),
            rW1, row(rb1), rW2, row(rb2), rW3, row(rb3), fcW, row(fcb),
            row(g1), row(be1), row(g2), row(be2), row(g3), row(be3),
            row(go), row(bo), unit)
    out = pl.pallas_call(
        _body,
        grid=(grid,),
        in_specs=[pl.BlockSpec((R, _F), lambda i: (i, 0))]
                 + [full(a) for a in args],
        out_specs=pl.BlockSpec((R, 4), lambda i: (i, 0)),
        out_shape=jax.ShapeDtypeStruct((B * _N, 4), f32),
        scratch_shapes=[pltpu.VMEM((R, R), f32)],
    )(state.astype(f32), *args)
    return out.reshape(B, _N * 4)


# speedup vs baseline: 1.4056x; 1.4056x over previous
import jax
import jax.numpy as jnp
from jax import lax
from jax.experimental import pallas as pl
from jax.experimental.pallas import tpu as pltpu
from jax.experimental.pallas import tpu_sc as plsc

_N = 64
_F = 16
_BG = 32


def _sc_body(e_hbm, c_hbm, deg_hbm, e_v, c_v, deg_v):
    cid = lax.axis_index("c")
    sid = lax.axis_index("s")

    @pl.when((cid == 0) & (sid == 0))
    def _():
        E = e_hbm.shape[1]
        pltpu.sync_copy(e_hbm, e_v)
        zeros = jnp.zeros((16,), jnp.float32)
        ones = jnp.ones((16,), jnp.float32)
        lane = lax.broadcasted_iota(jnp.int32, (16,), 0)

        def _zero(k, carry):
            c_v[pl.ds(k * 16, 16)] = zeros
            return carry
        lax.fori_loop(0, 4096 // 16, _zero, 0)
        for q in range(4):
            deg_v[pl.ds(q * 16, 16)] = zeros

        def _edges(t, carry):
            s = e_v[0, pl.ds(t * 16, 16)]
            d = e_v[1, pl.ds(t * 16, 16)]
            f = d * _N + s
            for j in range(16):
                m = lane == j
                plsc.addupdate_scatter(c_v, [f], ones, mask=m)
                plsc.addupdate_scatter(deg_v, [d], ones, mask=m)
            return carry
        lax.fori_loop(0, E // 16, _edges, 0)
        pltpu.sync_copy(c_v, c_hbm)
        pltpu.sync_copy(deg_v, deg_hbm)


def _sc_build(edge_index):
    import functools
    mesh = plsc.VectorSubcoreMesh(core_axis_name="c", subcore_axis_name="s")
    k = functools.partial(
        pl.kernel, mesh=mesh,
        out_type=(jax.ShapeDtypeStruct((_N * _N,), jnp.float32),
                  jax.ShapeDtypeStruct((_N,), jnp.float32)),
        scratch_types=[pltpu.VMEM(edge_index.shape, jnp.int32),
                       pltpu.VMEM((_N * _N,), jnp.float32),
                       pltpu.VMEM((_N,), jnp.float32)],
        compiler_params=pltpu.CompilerParams(needs_layout_passes=False),
    )(_sc_body)
    return k(edge_index)


def _dg_w(x3, W):
    return lax.dot_general(x3, W, (((2,), (0,)), ((), ())),
                           preferred_element_type=jnp.float32)


def _dg_mix(A, x3):
    return lax.dot_general(A, x3, (((1,), (0,)), ((), ())),
                           preferred_element_type=jnp.float32)


def _ln_lrelu(h, g, be):
    mu = jnp.mean(h, axis=2, keepdims=True)
    var = jnp.mean((h - mu) ** 2, axis=2, keepdims=True)
    hn = (h - mu) * lax.rsqrt(var + 1e-5) * g + be
    return jnp.where(hn > 0, hn, 0.01 * hn)


def _body(x_ref, c_ref, dr_ref, W1r, b1r, W2r, b2r, W3r, b3r,
          rW1r, rb1r, rW2r, rb2r, rW3r, rb3r, fcWr, fcbr,
          g1r, be1r, g2r, be2r, g3r, be3r, gor, bor, unit_ref,
          out_ref, A_ref):
    i = pl.program_id(0)

    @pl.when(i == 0)
    def _build():
        f32 = jnp.float32
        r64 = lax.broadcasted_iota(jnp.int32, (_N, _N), 0)
        c64 = lax.broadcasted_iota(jnp.int32, (_N, _N), 1)
        eye = (r64 == c64).astype(f32)
        C = c_ref[...] + eye
        deg_r = dr_ref[...] + 1.0
        deg_c = jnp.sum(C, axis=1, keepdims=True)
        A_ref[...] = C * lax.rsqrt(deg_c) * lax.rsqrt(deg_r)

    A = A_ref[...]
    x = x_ref[...]
    h1 = (_dg_mix(A, _dg_w(x, W1r[...])) + b1r[...]
          + _dg_w(x, rW1r[...]) + rb1r[...])
    x1 = _ln_lrelu(h1, g1r[...], be1r[...])
    h2 = (_dg_w(_dg_mix(A, x1), W2r[...]) + b2r[...]
          + _dg_w(x1, rW2r[...]) + rb2r[...])
    x2 = _ln_lrelu(h2, g2r[...], be2r[...])
    h3 = (_dg_w(_dg_mix(A, x2), W3r[...]) + b3r[...]
          + _dg_w(x2, rW3r[...]) + rb3r[...])
    x3 = _ln_lrelu(h3, g3r[...], be3r[...])
    h4 = _dg_w(x3, fcWr[0:256, :]) + _dg_w(x, fcWr[256:272, :]) + fcbr[...]
    mu = jnp.mean(h4, axis=2, keepdims=True)
    var = jnp.mean((h4 - mu) ** 2, axis=2, keepdims=True)
    hn = (h4 - mu) * lax.rsqrt(var + 1e-5) * gor[...] + bor[...]
    out_ref[...] = jnp.tanh(hn) * unit_ref[...]


def kernel(state, edge_index, batch_size, rej_rate, theta_value,
           W1, b1, W2, b2, W3, b3, rW1, rb1, rW2, rb2, rW3, rb3, fcW, fcb,
           g1, be1, g2, be2, g3, be3, go, bo):
    B = state.shape[0] // _N
    grid = B // _BG
    f32 = jnp.float32
    c_flat, deg = _sc_build(edge_index.astype(jnp.int32))
    c2d = c_flat.reshape(_N, _N)
    deg_row = deg.reshape(1, _N)
    unit = (jnp.asarray(batch_size).astype(f32) / jnp.asarray(B, f32)
            ).reshape(1, 1, 1)
    xnm = state.astype(f32).reshape(B, _N, _F).transpose(1, 0, 2)
    row = lambda v: v.reshape(1, 1, -1).astype(f32)
    full = lambda a: pl.BlockSpec(a.shape, lambda i: (0,) * a.ndim)
    args = (c2d, deg_row, W1, row(b1), W2, row(b2), W3, row(b3),
            rW1, row(rb1), rW2, row(rb2), rW3, row(rb3), fcW, row(fcb),
            row(g1), row(be1), row(g2), row(be2), row(g3), row(be3),
            row(go), row(bo), unit)
    out = pl.pallas_call(
        _body,
        grid=(grid,),
        in_specs=[pl.BlockSpec((_N, _BG, _F), lambda i: (0, i, 0))]
                 + [full(a) for a in args],
        out_specs=pl.BlockSpec((_N, _BG, 4), lambda i: (0, i, 0)),
        out_shape=jax.ShapeDtypeStruct((_N, B, 4), f32),
        scratch_shapes=[pltpu.VMEM((_N, _N), f32)],
    )(xnm, *args)
    return out.transpose(1, 0, 2).reshape(B, _N * 4)


# speedup vs baseline: 1.4262x; 1.0146x over previous
import jax
import jax.numpy as jnp
from jax import lax
from jax.experimental import pallas as pl
from jax.experimental.pallas import tpu as pltpu
from jax.experimental.pallas import tpu_sc as plsc

_N = 64
_F = 16
_BG = 64


def _sc_body(e_hbm, c_hbm, deg_hbm, e_v, c_v, deg_v):
    cid = lax.axis_index("c")
    sid = lax.axis_index("s")

    @pl.when((cid == 0) & (sid == 0))
    def _():
        E = e_hbm.shape[1]
        pltpu.sync_copy(e_hbm, e_v)
        zeros = jnp.zeros((16,), jnp.float32)
        ones = jnp.ones((16,), jnp.float32)
        lane = lax.broadcasted_iota(jnp.int32, (16,), 0)

        def _zero(k, carry):
            c_v[pl.ds(k * 16, 16)] = zeros
            return carry
        lax.fori_loop(0, 4096 // 16, _zero, 0)
        for q in range(4):
            deg_v[pl.ds(q * 16, 16)] = zeros

        def _edges(t, carry):
            s = e_v[0, pl.ds(t * 16, 16)]
            d = e_v[1, pl.ds(t * 16, 16)]
            f = d * _N + s
            for j in range(16):
                m = lane == j
                plsc.addupdate_scatter(c_v, [f], ones, mask=m)
                plsc.addupdate_scatter(deg_v, [d], ones, mask=m)
            return carry
        lax.fori_loop(0, E // 16, _edges, 0)
        pltpu.sync_copy(c_v, c_hbm)
        pltpu.sync_copy(deg_v, deg_hbm)


def _sc_build(edge_index):
    import functools
    mesh = plsc.VectorSubcoreMesh(core_axis_name="c", subcore_axis_name="s")
    k = functools.partial(
        pl.kernel, mesh=mesh,
        out_type=(jax.ShapeDtypeStruct((_N * _N,), jnp.float32),
                  jax.ShapeDtypeStruct((_N,), jnp.float32)),
        scratch_types=[pltpu.VMEM(edge_index.shape, jnp.int32),
                       pltpu.VMEM((_N * _N,), jnp.float32),
                       pltpu.VMEM((_N,), jnp.float32)],
        compiler_params=pltpu.CompilerParams(needs_layout_passes=False),
    )(_sc_body)
    return k(edge_index)


def _dg_w(x3, W):
    return lax.dot_general(x3, W, (((2,), (0,)), ((), ())),
                           preferred_element_type=jnp.float32)


def _dg_mix(A, x3):
    return lax.dot_general(A, x3, (((1,), (0,)), ((), ())),
                           preferred_element_type=jnp.float32)


def _ln_lrelu(h, g, be):
    mu = jnp.mean(h, axis=2, keepdims=True)
    var = jnp.mean((h - mu) ** 2, axis=2, keepdims=True)
    hn = (h - mu) * lax.rsqrt(var + 1e-5) * g + be
    return jnp.where(hn > 0, hn, 0.01 * hn)


def _body(x_ref, c_ref, dr_ref, Wc1r, bs1r, Wc2r, bs2r, Wc3r, bs3r,
          fcWr, fcbr, g1r, be1r, g2r, be2r, g3r, be3r, gor, bor, unit_ref,
          out_ref, AI_ref):
    i = pl.program_id(0)

    @pl.when(i == 0)
    def _build():
        f32 = jnp.float32
        r64 = lax.broadcasted_iota(jnp.int32, (_N, _N), 0)
        c64 = lax.broadcasted_iota(jnp.int32, (_N, _N), 1)
        eye = (r64 == c64).astype(f32)
        C = c_ref[...] + eye
        deg_r = dr_ref[...] + 1.0
        deg_c = jnp.sum(C, axis=1, keepdims=True)
        AI_ref[0:_N, :] = C * lax.rsqrt(deg_c) * lax.rsqrt(deg_r)
        AI_ref[_N:2 * _N, :] = eye

    AI = AI_ref[...]
    A = AI[0:_N, :]
    xb = x_ref[...]
    mx = lax.dot_general(AI, xb, (((1,), (1,)), ((), ())),
                         preferred_element_type=jnp.float32)
    x = mx[_N:2 * _N]
    h1 = _dg_w(jnp.concatenate([mx[0:_N], x], axis=2), Wc1r[...]) + bs1r[...]
    x1 = _ln_lrelu(h1, g1r[...], be1r[...])
    h2 = _dg_w(jnp.concatenate([_dg_mix(A, x1), x1], axis=2),
               Wc2r[...]) + bs2r[...]
    x2 = _ln_lrelu(h2, g2r[...], be2r[...])
    h3 = _dg_w(jnp.concatenate([_dg_mix(A, x2), x2], axis=2),
               Wc3r[...]) + bs3r[...]
    x3 = _ln_lrelu(h3, g3r[...], be3r[...])
    h4 = (_dg_w(x3, fcWr[0:4 * _N, :])
          + _dg_w(x, fcWr[4 * _N:4 * _N + _F, :]) + fcbr[...])
    mu = jnp.mean(h4, axis=2, keepdims=True)
    var = jnp.mean((h4 - mu) ** 2, axis=2, keepdims=True)
    hn = (h4 - mu) * lax.rsqrt(var + 1e-5) * gor[...] + bor[...]
    out_ref[...] = jnp.tanh(hn) * unit_ref[...]


def kernel(state, edge_index, batch_size, rej_rate, theta_value,
           W1, b1, W2, b2, W3, b3, rW1, rb1, rW2, rb2, rW3, rb3, fcW, fcb,
           g1, be1, g2, be2, g3, be3, go, bo):
    B = state.shape[0] // _N
    grid = B // _BG
    f32 = jnp.float32
    c_flat, deg = _sc_build(edge_index.astype(jnp.int32))
    c2d = c_flat.reshape(_N, _N)
    deg_row = deg.reshape(1, _N)
    unit = (jnp.asarray(batch_size).astype(f32) / jnp.asarray(B, f32)
            ).reshape(1, 1, 1)
    xb3 = state.astype(f32).reshape(B, _N, _F)
    row = lambda v: v.reshape(1, 1, -1).astype(f32)
    full = lambda a: pl.BlockSpec(a.shape, lambda i: (0,) * a.ndim)
    Wc1 = jnp.concatenate([W1, rW1], axis=0)
    Wc2 = jnp.concatenate([W2, rW2], axis=0)
    Wc3 = jnp.concatenate([W3, rW3], axis=0)
    args = (c2d, deg_row, Wc1, row(b1 + rb1), Wc2, row(b2 + rb2),
            Wc3, row(b3 + rb3), fcW, row(fcb),
            row(g1), row(be1), row(g2), row(be2), row(g3), row(be3),
            row(go), row(bo), unit)
    out = pl.pallas_call(
        _body,
        grid=(grid,),
        in_specs=[pl.BlockSpec((_BG, _N, _F), lambda i: (i, 0, 0))]
                 + [full(a) for a in args],
        out_specs=pl.BlockSpec((_N, _BG, 4), lambda i: (0, i, 0)),
        out_shape=jax.ShapeDtypeStruct((_N, B, 4), f32),
        scratch_shapes=[pltpu.VMEM((2 * _N, _N), f32)],
    )(xb3, *args)
    return out.transpose(1, 0, 2).reshape(B, _N * 4)


# speedup vs baseline: 1.5471x; 1.0848x over previous
import jax
import jax.numpy as jnp
from jax import lax
from jax.experimental import pallas as pl
from jax.experimental.pallas import tpu as pltpu
from jax.experimental.pallas import tpu_sc as plsc

_N = 64
_F = 16
_H = 64
_BG = 64


def _sc_body(e_hbm, c_hbm, deg_hbm, e_v, c_v, deg_v):
    cid = lax.axis_index("c")
    sid = lax.axis_index("s")

    @pl.when((cid == 0) & (sid == 0))
    def _():
        E = e_hbm.shape[1]
        pltpu.sync_copy(e_hbm, e_v)
        zeros = jnp.zeros((16,), jnp.float32)
        ones = jnp.ones((16,), jnp.float32)
        zeros_i = jnp.zeros((16,), jnp.int32)
        lane = lax.broadcasted_iota(jnp.int32, (16,), 0)

        def _zero(k, carry):
            c_v[k >> 2, pl.ds((k & 3) * 16, 16)] = zeros
            return carry
        lax.fori_loop(0, 4 * _N, _zero, 0)
        for q in range(4):
            deg_v[0, pl.ds(q * 16, 16)] = zeros

        def _edges(t, carry):
            s = e_v[0, pl.ds(t * 16, 16)]
            d = e_v[1, pl.ds(t * 16, 16)]
            for j in range(16):
                m = lane == j
                plsc.addupdate_scatter(c_v, [d, s], ones, mask=m)
                plsc.addupdate_scatter(deg_v, [zeros_i, d], ones, mask=m)
            return carry
        lax.fori_loop(0, E // 16, _edges, 0)
        pltpu.sync_copy(c_v, c_hbm)
        pltpu.sync_copy(deg_v, deg_hbm)


def _sc_build(edge_index):
    import functools
    mesh = plsc.VectorSubcoreMesh(core_axis_name="c", subcore_axis_name="s")
    k = functools.partial(
        pl.kernel, mesh=mesh,
        out_type=(jax.ShapeDtypeStruct((_N, _N), jnp.float32),
                  jax.ShapeDtypeStruct((1, _N), jnp.float32)),
        scratch_types=[pltpu.VMEM(edge_index.shape, jnp.int32),
                       pltpu.VMEM((_N, _N), jnp.float32),
                       pltpu.VMEM((1, _N), jnp.float32)],
        compiler_params=pltpu.CompilerParams(needs_layout_passes=False),
    )(_sc_body)
    return k(edge_index)


def _dg_w(x3, W):
    return lax.dot_general(x3, W, (((2,), (0,)), ((), ())),
                           preferred_element_type=jnp.float32)


def _dg_mix(A, x3):
    return lax.dot_general(A, x3, (((1,), (0,)), ((), ())),
                           preferred_element_type=jnp.float32)


def _ln_lrelu(h, g, be):
    mu = jnp.mean(h, axis=2, keepdims=True)
    var = jnp.mean((h - mu) ** 2, axis=2, keepdims=True)
    hn = (h - mu) * lax.rsqrt(var + 1e-5) * g + be
    return jnp.maximum(hn, 0.01 * hn)


def _body(x_ref, c_ref, dr_ref, W1r, b1r, W2r, b2r, W3r, b3r,
          rW1r, rb1r, rW2r, rb2r, rW3r, rb3r, fcWr, fcbr,
          g1r, be1r, g2r, be2r, g3r, be3r, gor, bor, unit_ref,
          out_ref, AI_ref, Wc1_ref, Wc2_ref, Wc3_ref):
    i = pl.program_id(0)

    @pl.when(i == 0)
    def _build():
        f32 = jnp.float32
        r64 = lax.broadcasted_iota(jnp.int32, (_N, _N), 0)
        c64 = lax.broadcasted_iota(jnp.int32, (_N, _N), 1)
        eye = (r64 == c64).astype(f32)
        C = c_ref[...] + eye
        deg_r = dr_ref[...] + 1.0
        deg_c = jnp.sum(C, axis=1, keepdims=True)
        AI_ref[0:_N, :] = C * lax.rsqrt(deg_c) * lax.rsqrt(deg_r)
        AI_ref[_N:2 * _N, :] = eye
        Wc1_ref[0:_F, :] = W1r[...]
        Wc1_ref[_F:2 * _F, :] = rW1r[...]
        Wc2_ref[0:_H, :] = W2r[...]
        Wc2_ref[_H:2 * _H, :] = rW2r[...]
        Wc3_ref[0:2 * _H, :] = W3r[...]
        Wc3_ref[2 * _H:4 * _H, :] = rW3r[...]

    AI = AI_ref[...]
    A = AI[0:_N, :]
    xb = x_ref[...]
    mx = lax.dot_general(AI, xb, (((1,), (1,)), ((), ())),
                         preferred_element_type=jnp.float32)
    x = mx[_N:2 * _N]
    h1 = (_dg_w(jnp.concatenate([mx[0:_N], x], axis=2), Wc1_ref[...])
          + b1r[...] + rb1r[...])
    x1 = _ln_lrelu(h1, g1r[...], be1r[...])
    h2 = (_dg_w(jnp.concatenate([_dg_mix(A, x1), x1], axis=2), Wc2_ref[...])
          + b2r[...] + rb2r[...])
    x2 = _ln_lrelu(h2, g2r[...], be2r[...])
    h3 = (_dg_w(jnp.concatenate([_dg_mix(A, x2), x2], axis=2), Wc3_ref[...])
          + b3r[...] + rb3r[...])
    x3 = _ln_lrelu(h3, g3r[...], be3r[...])
    h4 = (_dg_w(x3, fcWr[0:4 * _H, :])
          + _dg_w(x, fcWr[4 * _H:4 * _H + _F, :]) + fcbr[...])
    mu = jnp.mean(h4, axis=2, keepdims=True)
    var = jnp.mean((h4 - mu) ** 2, axis=2, keepdims=True)
    hn = (h4 - mu) * lax.rsqrt(var + 1e-5) * gor[...] + bor[...]
    out_ref[...] = jnp.tanh(hn) * unit_ref[...]


def kernel(state, edge_index, batch_size, rej_rate, theta_value,
           W1, b1, W2, b2, W3, b3, rW1, rb1, rW2, rb2, rW3, rb3, fcW, fcb,
           g1, be1, g2, be2, g3, be3, go, bo):
    B = state.shape[0] // _N
    grid = B // _BG
    f32 = jnp.float32
    c2d, deg_row = _sc_build(edge_index.astype(jnp.int32))
    unit = (jnp.asarray(batch_size).astype(f32) / jnp.asarray(B, f32)
            ).reshape(1, 1, 1)
    xb3 = state.astype(f32).reshape(B, _N, _F)
    row = lambda v: v.reshape(1, 1, -1).astype(f32)
    full = lambda a: pl.BlockSpec(a.shape, lambda i: (0,) * a.ndim)
    args = (c2d, deg_row, W1, row(b1), W2, row(b2), W3, row(b3),
            rW1, row(rb1), rW2, row(rb2), rW3, row(rb3), fcW, row(fcb),
            row(g1), row(be1), row(g2), row(be2), row(g3), row(be3),
            row(go), row(bo), unit)
    out = pl.pallas_call(
        _body,
        grid=(grid,),
        in_specs=[pl.BlockSpec((_BG, _N, _F), lambda i: (i, 0, 0))]
                 + [full(a) for a in args],
        out_specs=pl.BlockSpec((_N, _BG, 4), lambda i: (0, i, 0)),
        out_shape=jax.ShapeDtypeStruct((_N, B, 4), f32),
        scratch_shapes=[pltpu.VMEM((2 * _N, _N), f32),
                        pltpu.VMEM((2 * _F, _H), f32),
                        pltpu.VMEM((2 * _H, 2 * _H), f32),
                        pltpu.VMEM((4 * _H, 4 * _H), f32)],
    )(xb3, *args)
    return out.transpose(1, 0, 2).reshape(B, _N * 4)


# speedup vs baseline: 1.5825x; 1.0229x over previous
import jax
import jax.numpy as jnp
from jax import lax
from jax.experimental import pallas as pl
from jax.experimental.pallas import tpu as pltpu
from jax.experimental.pallas import tpu_sc as plsc

_N = 64
_F = 16
_H = 64
_BG = 64


def _sc_body(e_hbm, c_hbm, deg_hbm, e_v, c_v, deg_v):
    cid = lax.axis_index("c")
    sid = lax.axis_index("s")

    @pl.when((cid == 0) & (sid == 0))
    def _():
        E = e_hbm.shape[1]
        pltpu.sync_copy(e_hbm, e_v)
        zeros = jnp.zeros((16,), jnp.float32)
        ones = jnp.ones((16,), jnp.float32)
        zeros_i = jnp.zeros((16,), jnp.int32)
        lane = lax.broadcasted_iota(jnp.int32, (16,), 0)

        def _zero(k, carry):
            c_v[k >> 2, pl.ds((k & 3) * 16, 16)] = zeros
            return carry
        lax.fori_loop(0, 4 * _N, _zero, 0)
        for q in range(4):
            deg_v[0, pl.ds(q * 16, 16)] = zeros

        def _edges(t, carry):
            s = e_v[0, pl.ds(t * 16, 16)]
            d = e_v[1, pl.ds(t * 16, 16)]
            for j in range(16):
                m = lane == j
                plsc.addupdate_scatter(c_v, [d, s], ones, mask=m)
                plsc.addupdate_scatter(deg_v, [zeros_i, d], ones, mask=m)
            return carry
        lax.fori_loop(0, E // 16, _edges, 0)
        pltpu.sync_copy(c_v, c_hbm)
        pltpu.sync_copy(deg_v, deg_hbm)


def _sc_build(edge_index):
    import functools
    mesh = plsc.VectorSubcoreMesh(core_axis_name="c", subcore_axis_name="s")
    k = functools.partial(
        pl.kernel, mesh=mesh,
        out_type=(jax.ShapeDtypeStruct((_N, _N), jnp.float32),
                  jax.ShapeDtypeStruct((1, _N), jnp.float32)),
        scratch_types=[pltpu.VMEM(edge_index.shape, jnp.int32),
                       pltpu.VMEM((_N, _N), jnp.float32),
                       pltpu.VMEM((1, _N), jnp.float32)],
        compiler_params=pltpu.CompilerParams(needs_layout_passes=False),
    )(_sc_body)
    return k(edge_index)


def _dg_w(x3, W):
    return lax.dot_general(x3, W, (((2,), (0,)), ((), ())),
                           preferred_element_type=jnp.float32)


def _dg_mix(A, x3):
    return lax.dot_general(A, x3.astype(jnp.bfloat16),
                           (((1,), (0,)), ((), ())),
                           preferred_element_type=jnp.float32)


def _ln_lrelu(h, g, be):
    mu = jnp.mean(h, axis=2, keepdims=True)
    var = jnp.mean((h - mu) ** 2, axis=2, keepdims=True)
    hn = (h - mu) * lax.rsqrt(var + 1e-5) * g + be
    return jnp.maximum(hn, 0.01 * hn)


def _body(x_ref, c_ref, dr_ref, W1r, b1r, W2r, b2r, W3r, b3r,
          rW1r, rb1r, rW2r, rb2r, rW3r, rb3r, fcWr, fcbr,
          g1r, be1r, g2r, be2r, g3r, be3r, gor, bor, unit_ref,
          out_ref, AIb_ref, Wc1_ref, Wc2_ref, Wc3_ref):
    i = pl.program_id(0)

    @pl.when(i == 0)
    def _build():
        f32 = jnp.float32
        r64 = lax.broadcasted_iota(jnp.int32, (_N, _N), 0)
        c64 = lax.broadcasted_iota(jnp.int32, (_N, _N), 1)
        eye = (r64 == c64).astype(f32)
        C = c_ref[...] + eye
        deg_r = dr_ref[...] + 1.0
        deg_c = jnp.sum(C, axis=1, keepdims=True)
        A0 = C * lax.rsqrt(deg_c) * lax.rsqrt(deg_r)
        AIb_ref[0:_N, :] = A0.astype(jnp.bfloat16)
        AIb_ref[_N:2 * _N, :] = eye.astype(jnp.bfloat16)
        Wc1_ref[0:_F, :] = W1r[...]
        Wc1_ref[_F:2 * _F, :] = rW1r[...]
        Wc2_ref[0:_H, :] = W2r[...]
        Wc2_ref[_H:2 * _H, :] = rW2r[...]
        Wc3_ref[0:2 * _H, :] = W3r[...]
        Wc3_ref[2 * _H:4 * _H, :] = rW3r[...]

    AIb = AIb_ref[...]
    A = AIb[0:_N, :]
    xb = x_ref[...]
    mx = lax.dot_general(AIb, xb.astype(jnp.bfloat16), (((1,), (1,)), ((), ())),
                         preferred_element_type=jnp.float32)
    x = mx[_N:2 * _N]
    h1 = (_dg_w(jnp.concatenate([mx[0:_N], x], axis=2), Wc1_ref[...])
          + b1r[...] + rb1r[...])
    x1 = _ln_lrelu(h1, g1r[...], be1r[...])
    h2 = (_dg_w(jnp.concatenate([_dg_mix(A, x1), x1], axis=2), Wc2_ref[...])
          + b2r[...] + rb2r[...])
    x2 = _ln_lrelu(h2, g2r[...], be2r[...])
    h3 = (_dg_w(jnp.concatenate([_dg_mix(A, x2), x2], axis=2), Wc3_ref[...])
          + b3r[...] + rb3r[...])
    x3 = _ln_lrelu(h3, g3r[...], be3r[...])
    h4 = (_dg_w(x3, fcWr[0:4 * _H, :])
          + _dg_w(x, fcWr[4 * _H:4 * _H + _F, :]) + fcbr[...])
    mu = jnp.mean(h4, axis=2, keepdims=True)
    var = jnp.mean((h4 - mu) ** 2, axis=2, keepdims=True)
    hn = (h4 - mu) * lax.rsqrt(var + 1e-5) * gor[...] + bor[...]
    out_ref[...] = jnp.tanh(hn) * unit_ref[...]


def kernel(state, edge_index, batch_size, rej_rate, theta_value,
           W1, b1, W2, b2, W3, b3, rW1, rb1, rW2, rb2, rW3, rb3, fcW, fcb,
           g1, be1, g2, be2, g3, be3, go, bo):
    B = state.shape[0] // _N
    grid = B // _BG
    f32 = jnp.float32
    c2d, deg_row = _sc_build(edge_index.astype(jnp.int32))
    unit = (jnp.asarray(batch_size).astype(f32) / jnp.asarray(B, f32)
            ).reshape(1, 1, 1)
    xb3 = state.astype(f32).reshape(B, _N, _F)
    row = lambda v: v.reshape(1, 1, -1).astype(f32)
    full = lambda a: pl.BlockSpec(a.shape, lambda i: (0,) * a.ndim)
    args = (c2d, deg_row, W1, row(b1), W2, row(b2), W3, row(b3),
            rW1, row(rb1), rW2, row(rb2), rW3, row(rb3), fcW, row(fcb),
            row(g1), row(be1), row(g2), row(be2), row(g3), row(be3),
            row(go), row(bo), unit)
    out = pl.pallas_call(
        _body,
        grid=(grid,),
        in_specs=[pl.BlockSpec((_BG, _N, _F), lambda i: (i, 0, 0))]
                 + [full(a) for a in args],
        out_specs=pl.BlockSpec((_N, _BG, 4), lambda i: (0, i, 0)),
        out_shape=jax.ShapeDtypeStruct((_N, B, 4), f32),
        scratch_shapes=[pltpu.VMEM((2 * _N, _N), jnp.bfloat16),
                        pltpu.VMEM((2 * _F, _H), f32),
                        pltpu.VMEM((2 * _H, 2 * _H), f32),
                        pltpu.VMEM((4 * _H, 4 * _H), f32)],
    )(xb3, *args)
    return out.transpose(1, 0, 2).reshape(B, _N * 4)
